# Initial kernel scaffold; baseline (speedup 1.0000x reference)
import jax
import jax.numpy as jnp
from jax.experimental import pallas as pl


def kernel(x, edge_index, W, b):
    raise NotImplementedError("write your pallas kernel here")


# speedup vs baseline: 4.2069x; 4.2069x over previous
import jax
import jax.numpy as jnp
from jax import lax
from jax.experimental import pallas as pl
from jax.experimental.pallas import tpu as pltpu
from jax.experimental.pallas import tpu_sc as plsc

N_NODES = 10000
N_EDGES = 320000
FEATS = 128

NC = 2
NS = 16
CHUNK = 128
NCHUNK = 79
E_TILE = NCHUNK * CHUNK
E_PAD = NC * NS * E_TILE
ACC_ROWS = 10240
DUMMY_DST = N_NODES

ROWS_PER_TILE = ACC_ROWS // NS


def _tc_linear_relu(x_ref, w_ref, b_ref, h_ref):
    h = lax.dot_general(
        x_ref[...], w_ref[...], (((1,), (1,)), ((), ())),
        preferred_element_type=jnp.float32)
    h_ref[...] = jnp.maximum(h + b_ref[...], 0.0)


def _tc_add(a_ref, b_ref, o_ref):
    o_ref[...] = a_ref[...] + b_ref[...]


def _sc_body(h_hbm, src_hbm, dst_hbm, out0, out1,
             src_v, dst_v, rows_v, acc, sem):
    c = lax.axis_index("c")
    s = lax.axis_index("s")

    zeros16 = jnp.zeros((16,), jnp.float32)

    def _zero_row(i, _):
        for k in range(FEATS // 16):
            rows_v[i, pl.ds(k * 16, 16)] = zeros16
        return 0

    lax.fori_loop(0, CHUNK, _zero_row, 0)

    for k in range(ROWS_PER_TILE // CHUNK):
        pltpu.sync_copy(rows_v, acc.at[pl.ds(s * ROWS_PER_TILE + k * CHUNK, CHUNK)])
    plsc.subcore_barrier()

    base = (c * NS + s) * E_TILE

    def _chunk(j, _):
        e0 = base + j * CHUNK
        pltpu.sync_copy(src_hbm.at[pl.ds(e0, CHUNK)], src_v)
        pltpu.sync_copy(dst_hbm.at[pl.ds(e0, CHUNK)], dst_v.at[0])
        pltpu.async_copy(h_hbm.at[src_v], rows_v, sem).wait()
        pltpu.sync_copy(rows_v, acc.at[dst_v.at[0]], add=True)
        return 0

    lax.fori_loop(0, NCHUNK, _chunk, 0)
    plsc.subcore_barrier()

    def _writeback(out_hbm):
        for k in range(ROWS_PER_TILE // CHUNK):
            r0 = s * ROWS_PER_TILE + k * CHUNK
            pltpu.sync_copy(acc.at[pl.ds(r0, CHUNK)], rows_v)
            pltpu.sync_copy(rows_v, out_hbm.at[pl.ds(r0, CHUNK)])

    pl.when(c == 0)(lambda: _writeback(out0))
    pl.when(c == 1)(lambda: _writeback(out1))


@jax.jit
def kernel(x, edge_index, W, b):
    f32 = jnp.float32
    h = pl.pallas_call(
        _tc_linear_relu,
        grid=(10,),
        in_specs=[
            pl.BlockSpec((1000, FEATS), lambda i: (i, 0)),
            pl.BlockSpec((FEATS, FEATS), lambda i: (0, 0)),
            pl.BlockSpec((1, FEATS), lambda i: (0, 0)),
        ],
        out_specs=pl.BlockSpec((1000, FEATS), lambda i: (i, 0)),
        out_shape=jax.ShapeDtypeStruct((N_NODES, FEATS), f32),
    )(x, W, b.reshape(1, FEATS))

    src = edge_index[0].astype(jnp.int32)
    dst = edge_index[1].astype(jnp.int32)
    pad = E_PAD - N_EDGES
    src_p = jnp.concatenate([src, jnp.zeros((pad,), jnp.int32)])
    dst_p = jnp.concatenate([dst, jnp.full((pad,), DUMMY_DST, jnp.int32)])

    mesh = plsc.VectorSubcoreMesh(
        core_axis_name="c", subcore_axis_name="s",
        num_cores=NC, num_subcores=NS)
    p0, p1 = pl.kernel(
        _sc_body,
        out_type=(jax.ShapeDtypeStruct((ACC_ROWS, FEATS), f32),
                  jax.ShapeDtypeStruct((ACC_ROWS, FEATS), f32)),
        mesh=mesh,
        scratch_types=[
            pltpu.VMEM((CHUNK,), jnp.int32),
            pltpu.VMEM((1, CHUNK), jnp.int32),
            pltpu.VMEM((CHUNK, FEATS), f32),
            pltpu.VMEM_SHARED((ACC_ROWS, FEATS), f32),
            pltpu.SemaphoreType.DMA,
        ],
    )(h, src_p, dst_p)

    out = pl.pallas_call(
        _tc_add,
        grid=(10,),
        in_specs=[
            pl.BlockSpec((1000, FEATS), lambda i: (i, 0)),
            pl.BlockSpec((1000, FEATS), lambda i: (i, 0)),
        ],
        out_specs=pl.BlockSpec((1000, FEATS), lambda i: (i, 0)),
        out_shape=jax.ShapeDtypeStruct((N_NODES, FEATS), f32),
    )(p0[:N_NODES], p1[:N_NODES])
    return out
